# Initial kernel scaffold; baseline (speedup 1.0000x reference)
import jax
import jax.numpy as jnp
from jax.experimental import pallas as pl


def kernel(h, u, Wg, bg, Wb, bb, Wl, bl):
    raise NotImplementedError("write your pallas kernel here")


# speedup vs baseline: 5.8816x; 5.8816x over previous
import jax
import jax.numpy as jnp
from jax import lax
from jax.experimental import pallas as pl

EMB_D = 64
USR_D = 16
NEXP = 64
BLK = 2048


def _gate_body(h_ref, u_ref, wgt_ref, bg_ref, wbt_ref, bb_ref, wlt_ref,
               bl_ref, out_ref):
    u = u_ref[...]
    h = h_ref[...]
    gamma = jnp.dot(u, wgt_ref[...], preferred_element_type=jnp.float32)
    gamma = gamma + bg_ref[...]
    beta = jnp.dot(u, wbt_ref[...], preferred_element_type=jnp.float32)
    beta = beta + bb_ref[...]
    h_t = h * (1.0 + gamma) + beta
    logits = jnp.dot(h_t, wlt_ref[...], preferred_element_type=jnp.float32)
    logits = logits + bl_ref[...]

    iota = lax.broadcasted_iota(jnp.int32, logits.shape, 1)
    m1 = jnp.max(logits, axis=1, keepdims=True)
    eq1 = logits == m1
    idx1 = jnp.min(jnp.where(eq1, iota, NEXP), axis=1, keepdims=True)
    mask1 = iota == idx1
    l2 = jnp.where(mask1, -jnp.inf, logits)
    m2 = jnp.max(l2, axis=1, keepdims=True)
    eq2 = l2 == m2
    idx2 = jnp.min(jnp.where(eq2, iota, NEXP), axis=1, keepdims=True)
    mask2 = iota == idx2

    e = jnp.exp(m2 - m1)
    denom = 1.0 + e
    p1 = 1.0 / denom
    p2 = e / denom
    out_ref[...] = jnp.where(mask1, p1, jnp.where(mask2, p2, 0.0))


def kernel(h, u, Wg, bg, Wb, bb, Wl, bl):
    n = h.shape[0]
    grid = (n // BLK,)
    wgt = Wg.T
    wbt = Wb.T
    wlt = Wl.T
    bg2 = bg[None, :]
    bb2 = bb[None, :]
    bl2 = bl[None, :]
    return pl.pallas_call(
        _gate_body,
        grid=grid,
        in_specs=[
            pl.BlockSpec((BLK, EMB_D), lambda i: (i, 0)),
            pl.BlockSpec((BLK, USR_D), lambda i: (i, 0)),
            pl.BlockSpec((USR_D, EMB_D), lambda i: (0, 0)),
            pl.BlockSpec((1, EMB_D), lambda i: (0, 0)),
            pl.BlockSpec((USR_D, EMB_D), lambda i: (0, 0)),
            pl.BlockSpec((1, EMB_D), lambda i: (0, 0)),
            pl.BlockSpec((EMB_D, NEXP), lambda i: (0, 0)),
            pl.BlockSpec((1, NEXP), lambda i: (0, 0)),
        ],
        out_specs=pl.BlockSpec((BLK, NEXP), lambda i: (i, 0)),
        out_shape=jax.ShapeDtypeStruct((n, NEXP), jnp.float32),
    )(h, u, wgt, bg2, wbt, bb2, wlt, bl2)


# speedup vs baseline: 6.6312x; 1.1274x over previous
import jax
import jax.numpy as jnp
from jax import lax
from jax.experimental import pallas as pl

EMB_D = 64
USR_D = 16
NEXP = 64
BLK = 2048


def _gate_body(h_ref, u_ref, wgt_ref, bg_ref, wbt_ref, bb_ref, wlt_ref,
               bl_ref, out_ref):
    u = u_ref[...]
    h = h_ref[...]
    gamma = jnp.dot(u, wgt_ref[...], preferred_element_type=jnp.float32)
    gamma = gamma + bg_ref[...]
    beta = jnp.dot(u, wbt_ref[...], preferred_element_type=jnp.float32)
    beta = beta + bb_ref[...]
    h_t = h * (1.0 + gamma) + beta
    logits = jnp.dot(h_t, wlt_ref[...], preferred_element_type=jnp.float32)
    logits = logits + bl_ref[...]

    row = lax.broadcasted_iota(jnp.int32, (NEXP, NEXP), 0)
    col = lax.broadcasted_iota(jnp.int32, (NEXP, NEXP), 1)
    lt = (row <= col).astype(jnp.float32)

    m1 = jnp.max(logits, axis=1, keepdims=True)
    eq1 = logits == m1
    cs1 = jnp.dot(eq1.astype(jnp.float32), lt,
                  preferred_element_type=jnp.float32)
    mask1 = eq1 & (cs1 == 1.0)
    l2 = jnp.where(mask1, -jnp.inf, logits)
    m2 = jnp.max(l2, axis=1, keepdims=True)
    eq2 = l2 == m2
    cs2 = jnp.dot(eq2.astype(jnp.float32), lt,
                  preferred_element_type=jnp.float32)
    mask2 = eq2 & (cs2 == 1.0)

    e = jnp.exp(m2 - m1)
    denom = 1.0 + e
    p1 = 1.0 / denom
    p2 = e / denom
    out_ref[...] = jnp.where(mask1, p1, jnp.where(mask2, p2, 0.0))


def kernel(h, u, Wg, bg, Wb, bb, Wl, bl):
    n = h.shape[0]
    grid = (n // BLK,)
    wgt = Wg.T
    wbt = Wb.T
    wlt = Wl.T
    bg2 = bg[None, :]
    bb2 = bb[None, :]
    bl2 = bl[None, :]
    return pl.pallas_call(
        _gate_body,
        grid=grid,
        in_specs=[
            pl.BlockSpec((BLK, EMB_D), lambda i: (i, 0)),
            pl.BlockSpec((BLK, USR_D), lambda i: (i, 0)),
            pl.BlockSpec((USR_D, EMB_D), lambda i: (0, 0)),
            pl.BlockSpec((1, EMB_D), lambda i: (0, 0)),
            pl.BlockSpec((USR_D, EMB_D), lambda i: (0, 0)),
            pl.BlockSpec((1, EMB_D), lambda i: (0, 0)),
            pl.BlockSpec((EMB_D, NEXP), lambda i: (0, 0)),
            pl.BlockSpec((1, NEXP), lambda i: (0, 0)),
        ],
        out_specs=pl.BlockSpec((BLK, NEXP), lambda i: (i, 0)),
        out_shape=jax.ShapeDtypeStruct((n, NEXP), jnp.float32),
    )(h, u, wgt, bg2, wbt, bb2, wlt, bl2)


# speedup vs baseline: 6.9623x; 1.0499x over previous
import jax
import jax.numpy as jnp
from jax import lax
from jax.experimental import pallas as pl

EMB_D = 64
USR_D = 16
NEXP = 64
BLK = 4096


def _gate_body(h_ref, u_ref, wgt_ref, bg_ref, wbt_ref, bb_ref, wlt_ref,
               bl_ref, out_ref):
    u = u_ref[...]
    h = h_ref[...]
    gamma = jnp.dot(u, wgt_ref[...], preferred_element_type=jnp.float32)
    gamma = gamma + bg_ref[...]
    beta = jnp.dot(u, wbt_ref[...], preferred_element_type=jnp.float32)
    beta = beta + bb_ref[...]
    h_t = h * (1.0 + gamma) + beta
    logits = jnp.dot(h_t, wlt_ref[...], preferred_element_type=jnp.float32)
    logits = logits + bl_ref[...]

    row = lax.broadcasted_iota(jnp.int32, (NEXP, NEXP), 0)
    col = lax.broadcasted_iota(jnp.int32, (NEXP, NEXP), 1)
    lt = (row <= col).astype(jnp.float32)

    m1 = jnp.max(logits, axis=1, keepdims=True)
    eq1 = logits == m1
    cs1 = jnp.dot(eq1.astype(jnp.float32), lt,
                  preferred_element_type=jnp.float32)
    mask1 = eq1 & (cs1 == 1.0)
    l2 = jnp.where(mask1, -jnp.inf, logits)
    m2 = jnp.max(l2, axis=1, keepdims=True)
    eq2 = l2 == m2
    cs2 = jnp.dot(eq2.astype(jnp.float32), lt,
                  preferred_element_type=jnp.float32)
    mask2 = eq2 & (cs2 == 1.0)

    e = jnp.exp(m2 - m1)
    denom = 1.0 + e
    p1 = 1.0 / denom
    p2 = e / denom
    out_ref[...] = jnp.where(mask1, p1, jnp.where(mask2, p2, 0.0))


def kernel(h, u, Wg, bg, Wb, bb, Wl, bl):
    n = h.shape[0]
    grid = (n // BLK,)
    wgt = Wg.T
    wbt = Wb.T
    wlt = Wl.T
    bg2 = bg[None, :]
    bb2 = bb[None, :]
    bl2 = bl[None, :]
    return pl.pallas_call(
        _gate_body,
        grid=grid,
        in_specs=[
            pl.BlockSpec((BLK, EMB_D), lambda i: (i, 0)),
            pl.BlockSpec((BLK, USR_D), lambda i: (i, 0)),
            pl.BlockSpec((USR_D, EMB_D), lambda i: (0, 0)),
            pl.BlockSpec((1, EMB_D), lambda i: (0, 0)),
            pl.BlockSpec((USR_D, EMB_D), lambda i: (0, 0)),
            pl.BlockSpec((1, EMB_D), lambda i: (0, 0)),
            pl.BlockSpec((EMB_D, NEXP), lambda i: (0, 0)),
            pl.BlockSpec((1, NEXP), lambda i: (0, 0)),
        ],
        out_specs=pl.BlockSpec((BLK, NEXP), lambda i: (i, 0)),
        out_shape=jax.ShapeDtypeStruct((n, NEXP), jnp.float32),
    )(h, u, wgt, bg2, wbt, bb2, wlt, bl2)
